# Initial kernel scaffold; baseline (speedup 1.0000x reference)
import jax
import jax.numpy as jnp
from jax.experimental import pallas as pl


def kernel(embedding_v, embedding_u, center_words, target_words, outer_words):
    raise NotImplementedError("write your pallas kernel here")


# speedup vs baseline: 1.7019x; 1.7019x over previous
import functools

import jax
import jax.numpy as jnp
from jax import lax
from jax.experimental import pallas as pl
from jax.experimental.pallas import tpu as pltpu
from jax.experimental.pallas import tpu_sc as plsc

B = 4096
D = 64
NEG = 20
NC = 2
NS = 16
NW = NC * NS
BPW = B // NW


def _sc_body(v_hbm, u_hbm, cidx_hbm, tidx_hbm, oidx_hbm,
             c_out, t_out, usum_out,
             cidx_v, tidx_v, oidx_v, crows, trows, obuf, rowids, usum_sp,
             sem_c, sem_t, sem_o0, sem_o1):
    wid = lax.axis_index("s") * NC + lax.axis_index("c")
    base = wid * BPW

    pltpu.sync_copy(cidx_hbm.at[pl.ds(base, BPW)], cidx_v)
    pltpu.sync_copy(tidx_hbm.at[pl.ds(base, BPW)], tidx_v)
    pltpu.sync_copy(oidx_hbm.at[wid], oidx_v)

    cdma = pltpu.async_copy(v_hbm.at[cidx_v], crows, sem_c)
    tdma = pltpu.async_copy(u_hbm.at[tidx_v], trows, sem_t)

    for j in range(BPW // 16):
        rowids[pl.ds(j * 16, 16)] = base + j * 16 + lax.iota(jnp.int32, 16)

    sems = (sem_o0, sem_o1)
    dmas = [
        pltpu.async_copy(u_hbm.at[oidx_v.at[0]], obuf.at[0], sems[0]),
        pltpu.async_copy(u_hbm.at[oidx_v.at[1]], obuf.at[1], sems[1]),
    ]
    for k in range(NEG):
        b = k % 2
        dmas[b].wait()
        if k == 0:
            pltpu.sync_copy(obuf.at[0], usum_sp.at[pl.ds(base, BPW)])
        else:
            pltpu.sync_copy(obuf.at[b], usum_sp.at[rowids], add=True)
        if k + 2 < NEG:
            dmas[b] = pltpu.async_copy(
                u_hbm.at[oidx_v.at[k + 2]], obuf.at[b], sems[b])

    cdma.wait()
    pltpu.sync_copy(crows, c_out.at[pl.ds(base, BPW)])
    tdma.wait()
    pltpu.sync_copy(trows, t_out.at[pl.ds(base, BPW)])
    pltpu.sync_copy(usum_sp.at[pl.ds(base, BPW)], usum_out.at[pl.ds(base, BPW)])


@jax.jit
def _sc_gather(v, u, cidx, tidx, oidx3):
    mesh = plsc.VectorSubcoreMesh(
        core_axis_name="c", subcore_axis_name="s",
        num_cores=NC, num_subcores=NS)
    f = pl.kernel(
        _sc_body,
        out_type=(
            jax.ShapeDtypeStruct((B, D), jnp.float32),
            jax.ShapeDtypeStruct((B, D), jnp.float32),
            jax.ShapeDtypeStruct((B, D), jnp.float32),
        ),
        mesh=mesh,
        compiler_params=pltpu.CompilerParams(use_tc_tiling_on_sc=False),
        scratch_types=[
            pltpu.VMEM((BPW,), jnp.int32),
            pltpu.VMEM((BPW,), jnp.int32),
            pltpu.VMEM((NEG, BPW), jnp.int32),
            pltpu.VMEM((BPW, D), jnp.float32),
            pltpu.VMEM((BPW, D), jnp.float32),
            pltpu.VMEM((2, BPW, D), jnp.float32),
            pltpu.VMEM((BPW,), jnp.int32),
            pltpu.VMEM_SHARED((B, D), jnp.float32),
            pltpu.SemaphoreType.DMA,
            pltpu.SemaphoreType.DMA,
            pltpu.SemaphoreType.DMA,
            pltpu.SemaphoreType.DMA,
        ],
    )
    return f(v, u, cidx, tidx, oidx3)


def _log_sigmoid(x):
    return jnp.minimum(x, 0.0) - jnp.log1p(jnp.exp(-jnp.abs(x)))


def _loss_body(c_ref, t_ref, us_ref, out_ref):
    c = c_ref[...]
    p = jnp.sum(c * t_ref[...], axis=1)
    n = jnp.sum(c * us_ref[...], axis=1)
    tot = jnp.sum(_log_sigmoid(p)) + jnp.sum(_log_sigmoid(n))
    out_ref[...] = jnp.full((1, 1), -tot / B, jnp.float32)


@jax.jit
def _loss(c_rows, t_rows, usum):
    return pl.pallas_call(
        _loss_body,
        out_shape=jax.ShapeDtypeStruct((1, 1), jnp.float32),
    )(c_rows, t_rows, usum)


def kernel(embedding_v, embedding_u, center_words, target_words, outer_words):
    cidx = center_words.reshape(B).astype(jnp.int32)
    tidx = target_words.reshape(B).astype(jnp.int32)
    oidx3 = outer_words.astype(jnp.int32).reshape(NW, BPW, NEG).transpose(0, 2, 1)
    c_rows, t_rows, usum = _sc_gather(
        embedding_v, embedding_u, cidx, tidx, oidx3)
    out = _loss(c_rows, t_rows, usum)
    return out[0, 0]
